# speedup vs baseline: 1.5167x; 1.5167x over previous
import functools

import jax
import jax.numpy as jnp
from jax import lax
from jax.experimental import pallas as pl
from jax.experimental.pallas import tpu as pltpu
from jax.experimental.pallas import tpu_sc as plsc

VOCAB = 30522
HIDDEN = 1024
MAX_POS = 512
BATCH = 32
SEQ = 512
EPS = 1e-12

N_TOKENS = BATCH * SEQ
NUM_WORKERS = 32
TOK_PER_W = N_TOKENS // NUM_WORKERS
CHUNK = 64
N_CHUNKS = TOK_PER_W // CHUNK


def _sc_gather_body(ids_hbm, table_hbm, out_hbm, idx_v, rows_v, sem):
    wid = lax.axis_index("s") * 2 + lax.axis_index("c")
    base = wid * TOK_PER_W
    for c in range(N_CHUNKS):
        off = base + c * CHUNK
        pltpu.sync_copy(ids_hbm.at[pl.ds(off, CHUNK)], idx_v)
        pltpu.async_copy(table_hbm.at[idx_v], rows_v, sem).wait()
        pltpu.sync_copy(rows_v, out_hbm.at[pl.ds(off, CHUNK)])


@jax.jit
def _sc_gather(ids, table):
    mesh = plsc.VectorSubcoreMesh(core_axis_name="c", subcore_axis_name="s")
    return pl.kernel(
        _sc_gather_body,
        out_type=jax.ShapeDtypeStruct((N_TOKENS, HIDDEN), jnp.float32),
        mesh=mesh,
        scratch_types=[
            pltpu.VMEM((CHUNK,), jnp.int32),
            pltpu.VMEM((CHUNK, HIDDEN), jnp.float32),
            pltpu.SemaphoreType.DMA,
        ],
    )(ids, table)


TOK_BLK = 256
N_BLKS = N_TOKENS // TOK_BLK
S_BLKS = SEQ // TOK_BLK


def _tc_ln_body(g_ref, pos_ref, tt_ref, type_ref, gam_ref, bet_ref, out_ref):
    x = g_ref[...] + pos_ref[...]
    t0 = type_ref[0:1, :]
    dt = type_ref[1:2, :] - t0
    tt = tt_ref[0, 0, :]
    x = x + t0 + tt[:, None] * dt
    mean = jnp.mean(x, axis=-1, keepdims=True)
    xc = x - mean
    var = jnp.mean(xc * xc, axis=-1, keepdims=True)
    y = xc * lax.rsqrt(var + EPS)
    out_ref[...] = y * gam_ref[...] + bet_ref[...]


@jax.jit
def _tc_ln(gathered, pos_table, ttf, type_table, gamma, beta):
    return pl.pallas_call(
        _tc_ln_body,
        grid=(N_BLKS,),
        in_specs=[
            pl.BlockSpec((TOK_BLK, HIDDEN), lambda i: (i, 0)),
            pl.BlockSpec((TOK_BLK, HIDDEN), lambda i: (i % S_BLKS, 0)),
            pl.BlockSpec((1, 1, TOK_BLK), lambda i: (i, 0, 0)),
            pl.BlockSpec((2, HIDDEN), lambda i: (0, 0)),
            pl.BlockSpec((1, HIDDEN), lambda i: (0, 0)),
            pl.BlockSpec((1, HIDDEN), lambda i: (0, 0)),
        ],
        out_specs=pl.BlockSpec((TOK_BLK, HIDDEN), lambda i: (i, 0)),
        out_shape=jax.ShapeDtypeStruct((N_TOKENS, HIDDEN), jnp.float32),
    )(gathered, pos_table, ttf, type_table, gamma, beta)


def kernel(input_ids, token_type_ids, word_table, pos_table, type_table,
           ln_gamma, ln_beta):
    ids = input_ids.reshape(-1).astype(jnp.int32)
    gathered = _sc_gather(ids, word_table)
    ttf = token_type_ids.reshape(N_BLKS, 1, TOK_BLK).astype(jnp.float32)
    out = _tc_ln(gathered, pos_table, ttf, type_table,
                 ln_gamma.reshape(1, HIDDEN), ln_beta.reshape(1, HIDDEN))
    return out.reshape(BATCH, SEQ, HIDDEN)


# speedup vs baseline: 1.6471x; 1.0860x over previous
import functools

import jax
import jax.numpy as jnp
from jax import lax
from jax.experimental import pallas as pl
from jax.experimental.pallas import tpu as pltpu
from jax.experimental.pallas import tpu_sc as plsc

VOCAB = 30522
HIDDEN = 1024
MAX_POS = 512
BATCH = 32
SEQ = 512
EPS = 1e-12

N_TOKENS = BATCH * SEQ
NUM_WORKERS = 32
TOK_PER_W = N_TOKENS // NUM_WORKERS
CHUNK = 32
N_CHUNKS = TOK_PER_W // CHUNK


def _sc_gather_body(ids_hbm, table_hbm, out_hbm,
                    idx0, idx1, rows0, rows1, gsem0, gsem1, wsem0, wsem1):
    wid = lax.axis_index("s") * 2 + lax.axis_index("c")
    base = wid * N_CHUNKS
    idx = (idx0, idx1)
    rows = (rows0, rows1)
    gsem = (gsem0, gsem1)
    wsem = (wsem0, wsem1)

    def gather_cd(c, b):
        return pltpu.make_async_copy(table_hbm.at[idx[b]], rows[b], gsem[b])

    def write_cd(c, b):
        return pltpu.make_async_copy(
            rows[b], out_hbm.at[pl.ds((base + c) * CHUNK, CHUNK)], wsem[b])

    def start_gather(c, b):
        pltpu.sync_copy(ids_hbm.at[base + c], idx[b])
        gather_cd(c, b).start()

    start_gather(0, 0)
    start_gather(1, 1)
    for c in range(N_CHUNKS):
        b = c % 2
        gather_cd(c, b).wait()
        write_cd(c, b).start()
        if c + 2 < N_CHUNKS:
            write_cd(c - 2, b).wait()
            start_gather(c + 2, b)
    for c in (N_CHUNKS - 2, N_CHUNKS - 1):
        write_cd(c, c % 2).wait()


@jax.jit
def _sc_gather(ids2d, table):
    mesh = plsc.VectorSubcoreMesh(core_axis_name="c", subcore_axis_name="s")
    return pl.kernel(
        _sc_gather_body,
        out_type=jax.ShapeDtypeStruct((N_TOKENS, HIDDEN), jnp.float32),
        mesh=mesh,
        scratch_types=[
            pltpu.VMEM((CHUNK,), jnp.int32),
            pltpu.VMEM((CHUNK,), jnp.int32),
            pltpu.VMEM((CHUNK, HIDDEN), jnp.float32),
            pltpu.VMEM((CHUNK, HIDDEN), jnp.float32),
            pltpu.SemaphoreType.DMA,
            pltpu.SemaphoreType.DMA,
            pltpu.SemaphoreType.DMA,
            pltpu.SemaphoreType.DMA,
        ],
    )(ids2d, table)


TOK_BLK = 256
N_BLKS = N_TOKENS // TOK_BLK
S_BLKS = SEQ // TOK_BLK


def _tc_ln_body(g_ref, pos_ref, tt_ref, type_ref, gam_ref, bet_ref, out_ref):
    x = g_ref[...] + pos_ref[...]
    t0 = type_ref[0:1, :]
    dt = type_ref[1:2, :] - t0
    tt = tt_ref[0, 0, :]
    x = x + t0 + tt[:, None] * dt
    mean = jnp.mean(x, axis=-1, keepdims=True)
    xc = x - mean
    var = jnp.mean(xc * xc, axis=-1, keepdims=True)
    y = xc * lax.rsqrt(var + EPS)
    out_ref[...] = y * gam_ref[...] + bet_ref[...]


@jax.jit
def _tc_ln(gathered, pos_table, ttf, type_table, gamma, beta):
    return pl.pallas_call(
        _tc_ln_body,
        grid=(S_BLKS, BATCH),
        in_specs=[
            pl.BlockSpec((TOK_BLK, HIDDEN), lambda i, j: (j * S_BLKS + i, 0)),
            pl.BlockSpec((TOK_BLK, HIDDEN), lambda i, j: (i, 0)),
            pl.BlockSpec((1, 1, TOK_BLK), lambda i, j: (j * S_BLKS + i, 0, 0)),
            pl.BlockSpec((2, HIDDEN), lambda i, j: (0, 0)),
            pl.BlockSpec((1, HIDDEN), lambda i, j: (0, 0)),
            pl.BlockSpec((1, HIDDEN), lambda i, j: (0, 0)),
        ],
        out_specs=pl.BlockSpec((TOK_BLK, HIDDEN), lambda i, j: (j * S_BLKS + i, 0)),
        out_shape=jax.ShapeDtypeStruct((N_TOKENS, HIDDEN), jnp.float32),
    )(gathered, pos_table, ttf, type_table, gamma, beta)


def kernel(input_ids, token_type_ids, word_table, pos_table, type_table,
           ln_gamma, ln_beta):
    ids2d = input_ids.reshape(N_TOKENS // CHUNK, CHUNK).astype(jnp.int32)
    gathered = _sc_gather(ids2d, word_table)
    ttf = token_type_ids.reshape(N_BLKS, 1, TOK_BLK).astype(jnp.float32)
    out = _tc_ln(gathered, pos_table, ttf, type_table,
                 ln_gamma.reshape(1, HIDDEN), ln_beta.reshape(1, HIDDEN))
    return out.reshape(BATCH, SEQ, HIDDEN)


# speedup vs baseline: 1.7848x; 1.0836x over previous
import functools

import jax
import jax.numpy as jnp
from jax import lax
from jax.experimental import pallas as pl
from jax.experimental.pallas import tpu as pltpu
from jax.experimental.pallas import tpu_sc as plsc

VOCAB = 30522
HIDDEN = 1024
MAX_POS = 512
BATCH = 32
SEQ = 512
EPS = 1e-12

N_TOKENS = BATCH * SEQ
NUM_WORKERS = 32
TOK_PER_W = N_TOKENS // NUM_WORKERS
CHUNK = 32
N_CHUNKS = TOK_PER_W // CHUNK


def _sc_gather_body(n_chunks, ids_hbm, table_hbm, out_hbm,
                    idx0, idx1, rows0, rows1, gsem0, gsem1, wsem0, wsem1):
    wid = lax.axis_index("s") * 2 + lax.axis_index("c")
    base = wid * n_chunks
    idx = (idx0, idx1)
    rows = (rows0, rows1)
    gsem = (gsem0, gsem1)
    wsem = (wsem0, wsem1)

    def gather_cd(b):
        return pltpu.make_async_copy(table_hbm.at[idx[b]], rows[b], gsem[b])

    def write_cd(c, b):
        return pltpu.make_async_copy(
            rows[b], out_hbm.at[pl.ds((base + c) * CHUNK, CHUNK)], wsem[b])

    def start_gather(c, b):
        pltpu.sync_copy(ids_hbm.at[base + c], idx[b])
        gather_cd(b).start()

    start_gather(0, 0)
    if n_chunks > 1:
        start_gather(1, 1)
    for c in range(n_chunks):
        b = c % 2
        gather_cd(b).wait()
        write_cd(c, b).start()
        if c + 2 < n_chunks:
            write_cd(c, b).wait()
            start_gather(c + 2, b)
    for c in range(max(0, n_chunks - 2), n_chunks):
        write_cd(c, c % 2).wait()


@jax.jit
def _sc_gather(ids2d, table):
    n_rows = ids2d.shape[0]
    n_chunks = n_rows // NUM_WORKERS
    mesh = plsc.VectorSubcoreMesh(core_axis_name="c", subcore_axis_name="s")
    return pl.kernel(
        functools.partial(_sc_gather_body, n_chunks),
        out_type=jax.ShapeDtypeStruct((n_rows * CHUNK, HIDDEN), jnp.float32),
        mesh=mesh,
        scratch_types=[
            pltpu.VMEM((CHUNK,), jnp.int32),
            pltpu.VMEM((CHUNK,), jnp.int32),
            pltpu.VMEM((CHUNK, HIDDEN), jnp.float32),
            pltpu.VMEM((CHUNK, HIDDEN), jnp.float32),
            pltpu.SemaphoreType.DMA,
            pltpu.SemaphoreType.DMA,
            pltpu.SemaphoreType.DMA,
            pltpu.SemaphoreType.DMA,
        ],
    )(ids2d, table)


TOK_BLK = 256
N_BLKS = N_TOKENS // TOK_BLK
S_BLKS = SEQ // TOK_BLK


def _tc_ln_body(g_ref, pos_ref, tt_ref, type_ref, gam_ref, bet_ref, out_ref):
    x = g_ref[...] + pos_ref[...]
    t0 = type_ref[0:1, :]
    dt = type_ref[1:2, :] - t0
    tt = tt_ref[0, 0, :]
    x = x + t0 + tt[:, None] * dt
    mean = jnp.mean(x, axis=-1, keepdims=True)
    xc = x - mean
    var = jnp.mean(xc * xc, axis=-1, keepdims=True)
    y = xc * lax.rsqrt(var + EPS)
    out_ref[...] = y * gam_ref[...] + bet_ref[...]


K_SLICES = 4
B_SLICE = BATCH // K_SLICES
SLICE_TOK = B_SLICE * SEQ
ROWS_PER_SLICE = SLICE_TOK // CHUNK


def _make_tc_ln(s):
    aliased = s > 0

    def body(*refs):
        if aliased:
            g_ref, pos_ref, tt_ref, type_ref, gam_ref, bet_ref, _, out_ref = refs
        else:
            g_ref, pos_ref, tt_ref, type_ref, gam_ref, bet_ref, out_ref = refs
        _tc_ln_body(g_ref, pos_ref, tt_ref, type_ref, gam_ref, bet_ref,
                    out_ref)

    blk0 = s * B_SLICE * S_BLKS
    in_specs = [
        pl.BlockSpec((TOK_BLK, HIDDEN), lambda i, j: (j * S_BLKS + i, 0)),
        pl.BlockSpec((TOK_BLK, HIDDEN), lambda i, j: (i, 0)),
        pl.BlockSpec((1, 1, TOK_BLK), lambda i, j: (j * S_BLKS + i, 0, 0)),
        pl.BlockSpec((2, HIDDEN), lambda i, j: (0, 0)),
        pl.BlockSpec((1, HIDDEN), lambda i, j: (0, 0)),
        pl.BlockSpec((1, HIDDEN), lambda i, j: (0, 0)),
    ]
    if aliased:
        in_specs.append(pl.BlockSpec(memory_space=pl.ANY))
    return pl.pallas_call(
        body,
        grid=(S_BLKS, B_SLICE),
        in_specs=in_specs,
        out_specs=pl.BlockSpec((TOK_BLK, HIDDEN),
                               lambda i, j: (blk0 + j * S_BLKS + i, 0)),
        out_shape=jax.ShapeDtypeStruct((N_TOKENS, HIDDEN), jnp.float32),
        input_output_aliases={6: 0} if aliased else {},
    )


@jax.jit
def _pipeline(ids2d, word_table, pos_table, ttf, type_table, gamma, beta):
    gs = [
        _sc_gather(
            lax.slice_in_dim(ids2d, s * ROWS_PER_SLICE,
                             (s + 1) * ROWS_PER_SLICE, axis=0),
            word_table)
        for s in range(K_SLICES)
    ]
    buf = None
    for s in range(K_SLICES):
        tt_s = lax.slice_in_dim(ttf, s * B_SLICE * S_BLKS,
                                (s + 1) * B_SLICE * S_BLKS, axis=0)
        args = (gs[s], pos_table, tt_s, type_table, gamma, beta)
        buf = _make_tc_ln(s)(*(args if buf is None else args + (buf,)))
    return buf


def kernel(input_ids, token_type_ids, word_table, pos_table, type_table,
           ln_gamma, ln_beta):
    ids2d = input_ids.reshape(N_TOKENS // CHUNK, CHUNK).astype(jnp.int32)
    ttf = token_type_ids.reshape(N_BLKS, 1, TOK_BLK).astype(jnp.float32)
    out = _pipeline(ids2d, word_table, pos_table, ttf, type_table,
                    ln_gamma.reshape(1, HIDDEN), ln_beta.reshape(1, HIDDEN))
    return out.reshape(BATCH, SEQ, HIDDEN)
